# Initial kernel scaffold; baseline (speedup 1.0000x reference)
import functools

import jax
import jax.numpy as jnp
from jax import lax
from jax.experimental import pallas as pl
from jax.experimental.pallas import tpu as pltpu
from jax.experimental.pallas import tpu_sc as plsc

B, R, N, K = 8, 32, 128, 64
ROWS = B * R
NC, NS, L = 2, 16, 16
NW = NC * NS
ROWS_PER_W = ROWS // NW
ELEMS_PER_W = ROWS_PER_W * K
VECS = ELEMS_PER_W // L
VECS_PER_ROW = K // L

_mesh = plsc.VectorSubcoreMesh(core_axis_name="c", subcore_axis_name="s")


@functools.partial(
    pl.kernel,
    mesh=_mesh,
    out_type=jax.ShapeDtypeStruct((ROWS * K,), jnp.float32),
    scratch_types=[
        pltpu.VMEM((ROWS_PER_W * N,), jnp.float32),
        pltpu.VMEM((ELEMS_PER_W,), jnp.int32),
        pltpu.VMEM((ELEMS_PER_W,), jnp.float32),
    ],
)
def _gather_sc(x_hbm, idx_hbm, out_hbm, x_v, idx_v, out_v):
    wid = lax.axis_index("s") * NC + lax.axis_index("c")
    x_base = wid * ROWS_PER_W * N
    e_base = wid * ELEMS_PER_W
    pltpu.sync_copy(x_hbm.at[pl.ds(x_base, ROWS_PER_W * N)], x_v)
    pltpu.sync_copy(idx_hbm.at[pl.ds(e_base, ELEMS_PER_W)], idx_v)
    for i in range(VECS):
        idx = idx_v[pl.ds(i * L, L)] + (i // VECS_PER_ROW) * N
        out_v[pl.ds(i * L, L)] = plsc.load_gather(x_v, [idx])
    pltpu.sync_copy(out_v, out_hbm.at[pl.ds(e_base, ELEMS_PER_W)])


def kernel(x, index, dim):
    del dim
    xf = x.reshape(-1).astype(jnp.float32)
    idxf = index.astype(jnp.int32).reshape(-1)
    out = _gather_sc(xf, idxf)
    return out.reshape(index.shape).astype(x.dtype)


# speedup vs baseline: 1.0374x; 1.0374x over previous
import functools

import jax
import jax.numpy as jnp
from jax import lax
from jax.experimental import pallas as pl
from jax.experimental.pallas import tpu as pltpu
from jax.experimental.pallas import tpu_sc as plsc

B, R, N, K = 8, 32, 128, 64
ROWS = B * R
NC, NS, L = 2, 16, 16
NW = NC * NS
ROWS_PER_W = ROWS // NW
ELEMS_PER_W = ROWS_PER_W * K
VECS = ELEMS_PER_W // L
VECS_PER_ROW = K // L

_mesh = plsc.VectorSubcoreMesh(core_axis_name="c", subcore_axis_name="s")


@functools.partial(
    pl.kernel,
    mesh=_mesh,
    out_type=jax.ShapeDtypeStruct((ROWS * K,), jnp.float32),
    scratch_types=[
        pltpu.VMEM((ROWS_PER_W * N,), jnp.float32),
        pltpu.VMEM((ELEMS_PER_W,), jnp.int32),
        pltpu.VMEM((ELEMS_PER_W,), jnp.float32),
    ],
    compiler_params=pltpu.CompilerParams(needs_layout_passes=False),
)
def _gather_sc(x_hbm, idx_hbm, out_hbm, x_v, idx_v, out_v):
    wid = lax.axis_index("s") * NC + lax.axis_index("c")
    x_base = wid * ROWS_PER_W * N
    e_base = wid * ELEMS_PER_W
    pltpu.sync_copy(x_hbm.at[pl.ds(x_base, ROWS_PER_W * N)], x_v)
    pltpu.sync_copy(idx_hbm.at[pl.ds(e_base, ELEMS_PER_W)], idx_v)
    for i in range(VECS):
        idx = idx_v[pl.ds(i * L, L)] + (i // VECS_PER_ROW) * N
        out_v[pl.ds(i * L, L)] = plsc.load_gather(x_v, [idx])
    pltpu.sync_copy(out_v, out_hbm.at[pl.ds(e_base, ELEMS_PER_W)])


def kernel(x, index, dim):
    del dim
    xf = x.reshape(-1).astype(jnp.float32)
    idxf = index.astype(jnp.int32).reshape(-1)
    out = _gather_sc(xf, idxf)
    return out.reshape(index.shape).astype(x.dtype)
